# baseline (speedup 1.0000x reference)
import functools

import jax
import jax.numpy as jnp
from jax import lax
from jax.experimental import pallas as pl
from jax.experimental.pallas import tpu as pltpu
from jax.experimental.pallas import tpu_sc as plsc

NUM_CORES = 2
NUM_SUBCORES = 16
LANES = 16
NW = NUM_CORES * NUM_SUBCORES
BATCH = 16384
DIM = 64
HDIM = DIM // 2
NROWS = 1000000
BPW = BATCH // NW
CHUNK = 128
NCHUNK = BPW // CHUNK
PASSES = 2
CPP = NCHUNK // PASSES
EPP = BPW // PASSES
TCW = 8192
GROUP = TCW // 4
NBLK = (NROWS + TCW - 1) // TCW
PROWS = NBLK * GROUP


def _pack_body(u_ref, i_ref, up_ref, ip_ref):
    for src, dst in ((u_ref, up_ref), (i_ref, ip_ref)):
        bits = lax.bitcast_convert_type(src[...], jnp.int32)
        rne = (bits + 0x7FFF + ((bits >> 16) & 1)) >> 16
        lo = rne[0:HDIM, :] & 0xFFFF
        hi = rne[HDIM:DIM, :] << 16
        packed = (hi | lo).T
        for h in range(4):
            dst[:, h * HDIM:(h + 1) * HDIM] = packed[
                h * GROUP:(h + 1) * GROUP, :]


def _pack(ut, it):
    out = jax.ShapeDtypeStruct((PROWS, 4 * HDIM), jnp.int32)
    return pl.pallas_call(
        _pack_body,
        grid=(NBLK,),
        in_specs=[
            pl.BlockSpec((DIM, TCW), lambda i: (0, i)),
            pl.BlockSpec((DIM, TCW), lambda i: (0, i)),
        ],
        out_specs=[
            pl.BlockSpec((GROUP, 4 * HDIM), lambda i: (i, 0)),
            pl.BlockSpec((GROUP, 4 * HDIM), lambda i: (i, 0)),
        ],
        out_shape=[out, out],
    )(ut, it)


def _make_sc_kernel():
    mesh = plsc.VectorSubcoreMesh(core_axis_name="c", subcore_axis_name="s")

    @functools.partial(
        pl.kernel,
        mesh=mesh,
        out_type=jax.ShapeDtypeStruct((BATCH,), jnp.float32),
        compiler_params=pltpu.CompilerParams(
            use_tc_tiling_on_sc=False, needs_layout_passes=False),
        scratch_types=[
            pltpu.VMEM((NCHUNK, CHUNK), jnp.int32),
            pltpu.VMEM((NCHUNK, CHUNK), jnp.int32),
            pltpu.VMEM((NCHUNK, CHUNK), jnp.int32),
            pltpu.VMEM((NCHUNK, CHUNK), jnp.int32),
            pltpu.VMEM((EPP, 4 * HDIM), jnp.int32),
            pltpu.VMEM((EPP, 4 * HDIM), jnp.int32),
            pltpu.VMEM((BPW,), jnp.float32),
            pltpu.SemaphoreType.DMA,
        ],
    )
    def lightgcn_dot(urow_hbm, irow_hbm, ugrp_hbm, igrp_hbm,
                     utab_hbm, itab_hbm, out_hbm,
                     urow_v, irow_v, ugrp_v, igrp_v,
                     urows_v, irows_v, out_v, sem):
        wid = lax.axis_index("s") * NUM_CORES + lax.axis_index("c")
        base = wid * BPW

        pltpu.sync_copy(urow_hbm.at[wid], urow_v)
        pltpu.sync_copy(irow_hbm.at[wid], irow_v)
        pltpu.sync_copy(ugrp_hbm.at[wid], ugrp_v)
        pltpu.sync_copy(igrp_hbm.at[wid], igrp_v)

        lane = lax.broadcasted_iota(jnp.int32, (LANES,), 0)
        perms = [lane ^ (1 << k) for k in range(4)]
        dnums = lax.GatherDimensionNumbers(
            offset_dims=(), collapsed_slice_dims=(0,), start_index_map=(0,))

        def _permute(v, idx):
            return lax.gather(
                v, idx[:, None], dnums, slice_sizes=(1,),
                mode=lax.GatherScatterMode.PROMISE_IN_BOUNDS)

        def _allsum(v):
            for p in perms:
                v = v + _permute(v, p)
            return v

        hmask = jnp.int32(-65536)

        for p in range(PASSES):
            copies = []
            for j in range(CPP):
                copies.append(pltpu.async_copy(
                    utab_hbm.at[urow_v.at[p * CPP + j]],
                    urows_v.at[pl.ds(j * CHUNK, CHUNK)], sem))
                copies.append(pltpu.async_copy(
                    itab_hbm.at[irow_v.at[p * CPP + j]],
                    irows_v.at[pl.ds(j * CHUNK, CHUNK)], sem))
            for c in copies:
                c.wait()

            def group_body(g, carry):
                base_row = g * LANES
                chunk_row = p * CPP + g // (CHUNK // LANES)
                chunk_off = (g % (CHUNK // LANES)) * LANES
                ug16 = ugrp_v[chunk_row, pl.ds(chunk_off, LANES)]
                ig16 = igrp_v[chunk_row, pl.ds(chunk_off, LANES)]
                res = jnp.zeros((LANES,), jnp.float32)
                for r in range(LANES):
                    b = base_row + r
                    rsplat = jnp.full((LANES,), r, jnp.int32)
                    ug = _permute(ug16, rsplat)
                    ig = _permute(ig16, rsplat)
                    acc = None
                    for c in range(HDIM // LANES):
                        ubits = None
                        vbits = None
                        for h in range(4):
                            cu = urows_v[b, pl.ds(h * HDIM + c * LANES, LANES)]
                            ci = irows_v[b, pl.ds(h * HDIM + c * LANES, LANES)]
                            if h == 0:
                                ubits, vbits = cu, ci
                            else:
                                ubits = jnp.where(ug == h, cu, ubits)
                                vbits = jnp.where(ig == h, ci, vbits)
                        ulo = plsc.bitcast(ubits << 16, jnp.float32)
                        uhi = plsc.bitcast(ubits & hmask, jnp.float32)
                        vlo = plsc.bitcast(vbits << 16, jnp.float32)
                        vhi = plsc.bitcast(vbits & hmask, jnp.float32)
                        prod = ulo * vlo + uhi * vhi
                        acc = prod if acc is None else acc + prod
                    res = jnp.where(lane == r, _allsum(acc), res)
                out_v[pl.ds(p * EPP + base_row, LANES)] = res
                return carry

            lax.fori_loop(0, EPP // LANES, group_body, 0)

        pltpu.sync_copy(out_v, out_hbm.at[pl.ds(base, BPW)])

    return lightgcn_dot


_SC_KERNEL = _make_sc_kernel()


def _row_grp(ids):
    row = (ids // TCW) * GROUP + (ids % GROUP)
    grp = (ids % TCW) // GROUP
    return row, grp


def kernel(user_ids, item_ids, user_table, item_table):
    uids = user_ids.astype(jnp.int32)
    iids = item_ids.astype(jnp.int32)
    urow, ugrp = _row_grp(uids)
    irow, igrp = _row_grp(iids)
    shape = (NW, NCHUNK, CHUNK)
    utab, itab = _pack(user_table.T, item_table.T)
    return _SC_KERNEL(urow.reshape(shape), irow.reshape(shape),
                      ugrp.reshape(shape), igrp.reshape(shape),
                      utab, itab)


# speedup vs baseline: 1.5451x; 1.5451x over previous
import functools

import jax
import jax.numpy as jnp
from jax import lax
from jax.experimental import pallas as pl
from jax.experimental.pallas import tpu as pltpu
from jax.experimental.pallas import tpu_sc as plsc

NUM_CORES = 2
NUM_SUBCORES = 16
LANES = 16
NW = NUM_CORES * NUM_SUBCORES
BATCH = 16384
DIM = 64
HDIM = DIM // 2
NROWS = 1000000
BPW = BATCH // NW
CHUNK = 128
NCHUNK = BPW // CHUNK
PASSES = 2
CPP = NCHUNK // PASSES
EPP = BPW // PASSES
TCW = 8192
HROW = TCW // 2
NBLK = (NROWS + TCW - 1) // TCW
PROWS = NBLK * HROW


def _pack_body(u_ref, i_ref, out_ref, xt_ref):
    packs = []
    for src in (u_ref, i_ref):
        bits = lax.bitcast_convert_type(src[...], jnp.int32)
        rne = (bits + 0x7FFF + ((bits >> 16) & 1)) >> 16
        packs.append((rne[HDIM:DIM, :] << 16) | (rne[0:HDIM, :] & 0xFFFF))
    cat = jnp.concatenate(packs, axis=0)
    xt_ref[...] = lax.bitcast_convert_type(cat, jnp.float32)
    catt = xt_ref[...].T
    out_ref[:, 0:DIM] = lax.bitcast_convert_type(
        catt[0:HROW, :], jnp.int32)
    out_ref[:, DIM:2 * DIM] = lax.bitcast_convert_type(
        catt[HROW:TCW, :], jnp.int32)


def _pack(ut, it):
    return pl.pallas_call(
        _pack_body,
        grid=(NBLK,),
        in_specs=[
            pl.BlockSpec((DIM, TCW), lambda i: (0, i)),
            pl.BlockSpec((DIM, TCW), lambda i: (0, i)),
        ],
        out_specs=pl.BlockSpec((HROW, 2 * DIM), lambda i: (i, 0)),
        out_shape=jax.ShapeDtypeStruct((PROWS, 2 * DIM), jnp.int32),
        scratch_shapes=[pltpu.VMEM((DIM, TCW), jnp.float32)],
    )(ut, it)


def _make_sc_kernel():
    mesh = plsc.VectorSubcoreMesh(core_axis_name="c", subcore_axis_name="s")

    @functools.partial(
        pl.kernel,
        mesh=mesh,
        out_type=jax.ShapeDtypeStruct((BATCH,), jnp.float32),
        compiler_params=pltpu.CompilerParams(
            use_tc_tiling_on_sc=False, needs_layout_passes=False),
        scratch_types=[
            pltpu.VMEM((NCHUNK, CHUNK), jnp.int32),
            pltpu.VMEM((NCHUNK, CHUNK), jnp.int32),
            pltpu.VMEM((NCHUNK, CHUNK), jnp.int32),
            pltpu.VMEM((NCHUNK, CHUNK), jnp.int32),
            pltpu.VMEM((EPP, 2 * DIM), jnp.int32),
            pltpu.VMEM((EPP, 2 * DIM), jnp.int32),
            pltpu.VMEM((BPW,), jnp.float32),
            pltpu.SemaphoreType.DMA,
        ],
    )
    def lightgcn_dot(urow_hbm, irow_hbm, uh_hbm, ih_hbm, tab_hbm, out_hbm,
                     urow_v, irow_v, uh_v, ih_v,
                     urows_v, irows_v, out_v, sem):
        wid = lax.axis_index("s") * NUM_CORES + lax.axis_index("c")
        base = wid * BPW

        pltpu.sync_copy(urow_hbm.at[wid], urow_v)
        pltpu.sync_copy(irow_hbm.at[wid], irow_v)
        pltpu.sync_copy(uh_hbm.at[wid], uh_v)
        pltpu.sync_copy(ih_hbm.at[wid], ih_v)

        lane = lax.broadcasted_iota(jnp.int32, (LANES,), 0)
        perms = [lane ^ (1 << k) for k in range(4)]
        dnums = lax.GatherDimensionNumbers(
            offset_dims=(), collapsed_slice_dims=(0,), start_index_map=(0,))

        def _permute(v, idx):
            return lax.gather(
                v, idx[:, None], dnums, slice_sizes=(1,),
                mode=lax.GatherScatterMode.PROMISE_IN_BOUNDS)

        def _allsum(v):
            for p in perms:
                v = v + _permute(v, p)
            return v

        hmask = jnp.int32(-65536)

        for p in range(PASSES):
            copies = []
            for j in range(CPP):
                copies.append(pltpu.async_copy(
                    tab_hbm.at[urow_v.at[p * CPP + j]],
                    urows_v.at[pl.ds(j * CHUNK, CHUNK)], sem))
                copies.append(pltpu.async_copy(
                    tab_hbm.at[irow_v.at[p * CPP + j]],
                    irows_v.at[pl.ds(j * CHUNK, CHUNK)], sem))
            for c in copies:
                c.wait()

            def group_body(g, carry):
                base_row = g * LANES
                chunk_row = p * CPP + g // (CHUNK // LANES)
                chunk_off = (g % (CHUNK // LANES)) * LANES
                uh16 = uh_v[chunk_row, pl.ds(chunk_off, LANES)]
                ih16 = ih_v[chunk_row, pl.ds(chunk_off, LANES)]
                res = jnp.zeros((LANES,), jnp.float32)
                for r in range(LANES):
                    b = base_row + r
                    rsplat = jnp.full((LANES,), r, jnp.int32)
                    uh = _permute(uh16, rsplat)
                    ih = _permute(ih16, rsplat)
                    acc = None
                    for c in range(HDIM // LANES):
                        u0 = urows_v[b, pl.ds(c * LANES, LANES)]
                        u1 = urows_v[b, pl.ds(DIM + c * LANES, LANES)]
                        i0 = irows_v[b, pl.ds(HDIM + c * LANES, LANES)]
                        i1 = irows_v[b, pl.ds(DIM + HDIM + c * LANES, LANES)]
                        ubits = jnp.where(uh == 1, u1, u0)
                        vbits = jnp.where(ih == 1, i1, i0)
                        ulo = plsc.bitcast(ubits << 16, jnp.float32)
                        uhi = plsc.bitcast(ubits & hmask, jnp.float32)
                        vlo = plsc.bitcast(vbits << 16, jnp.float32)
                        vhi = plsc.bitcast(vbits & hmask, jnp.float32)
                        prod = ulo * vlo + uhi * vhi
                        acc = prod if acc is None else acc + prod
                    res = jnp.where(lane == r, _allsum(acc), res)
                out_v[pl.ds(p * EPP + base_row, LANES)] = res
                return carry

            lax.fori_loop(0, EPP // LANES, group_body, 0)

        pltpu.sync_copy(out_v, out_hbm.at[pl.ds(base, BPW)])

    return lightgcn_dot


_SC_KERNEL = _make_sc_kernel()


def _row_half(ids):
    row = (ids // TCW) * HROW + (ids % HROW)
    half = (ids % TCW) // HROW
    return row, half


def kernel(user_ids, item_ids, user_table, item_table):
    uids = user_ids.astype(jnp.int32)
    iids = item_ids.astype(jnp.int32)
    urow, uh = _row_half(uids)
    irow, ih = _row_half(iids)
    shape = (NW, NCHUNK, CHUNK)
    tab = _pack(user_table.T, item_table.T)
    return _SC_KERNEL(urow.reshape(shape), irow.reshape(shape),
                      uh.reshape(shape), ih.reshape(shape), tab)


# speedup vs baseline: 1.7334x; 1.1219x over previous
import functools

import jax
import jax.numpy as jnp
from jax import lax
from jax.experimental import pallas as pl
from jax.experimental.pallas import tpu as pltpu
from jax.experimental.pallas import tpu_sc as plsc

NUM_CORES = 2
NUM_SUBCORES = 16
LANES = 16
NW = NUM_CORES * NUM_SUBCORES
BATCH = 16384
DIM = 64
HDIM = DIM // 2
NROWS = 1000000
BPW = BATCH // NW
CHUNK = 128
NCHUNK = BPW // CHUNK
PASSES = 2
CPP = NCHUNK // PASSES
EPP = BPW // PASSES
TCW = 16384
HROW = TCW // 2
NBLK = (NROWS + TCW - 1) // TCW
PROWS = NBLK * HROW


def _pack_body(u_ref, i_ref, out_ref, xt_ref):
    packs = []
    for src in (u_ref, i_ref):
        bits = lax.bitcast_convert_type(src[...], jnp.int32)
        rne = (bits + 0x7FFF + ((bits >> 16) & 1)) >> 16
        packs.append((rne[HDIM:DIM, :] << 16) | (rne[0:HDIM, :] & 0xFFFF))
    cat = jnp.concatenate(packs, axis=0)
    xt_ref[...] = lax.bitcast_convert_type(cat, jnp.float32)
    catt = xt_ref[...].T
    out_ref[:, 0:DIM] = lax.bitcast_convert_type(
        catt[0:HROW, :], jnp.int32)
    out_ref[:, DIM:2 * DIM] = lax.bitcast_convert_type(
        catt[HROW:TCW, :], jnp.int32)


def _pack(ut, it):
    return pl.pallas_call(
        _pack_body,
        grid=(NBLK,),
        in_specs=[
            pl.BlockSpec((DIM, TCW), lambda i: (0, i)),
            pl.BlockSpec((DIM, TCW), lambda i: (0, i)),
        ],
        out_specs=pl.BlockSpec((HROW, 2 * DIM), lambda i: (i, 0)),
        out_shape=jax.ShapeDtypeStruct((PROWS, 2 * DIM), jnp.int32),
        scratch_shapes=[pltpu.VMEM((DIM, TCW), jnp.float32)],
    )(ut, it)


def _make_sc_kernel():
    mesh = plsc.VectorSubcoreMesh(core_axis_name="c", subcore_axis_name="s")

    @functools.partial(
        pl.kernel,
        mesh=mesh,
        out_type=jax.ShapeDtypeStruct((BATCH,), jnp.float32),
        compiler_params=pltpu.CompilerParams(
            use_tc_tiling_on_sc=False, needs_layout_passes=False),
        scratch_types=[
            pltpu.VMEM((NCHUNK, CHUNK), jnp.int32),
            pltpu.VMEM((NCHUNK, CHUNK), jnp.int32),
            pltpu.VMEM((NCHUNK, CHUNK), jnp.int32),
            pltpu.VMEM((NCHUNK, CHUNK), jnp.int32),
            pltpu.VMEM((EPP, 2 * DIM), jnp.int32),
            pltpu.VMEM((EPP, 2 * DIM), jnp.int32),
            pltpu.VMEM((BPW,), jnp.float32),
            pltpu.SemaphoreType.DMA,
        ],
    )
    def lightgcn_dot(urow_hbm, irow_hbm, uh_hbm, ih_hbm, tab_hbm, out_hbm,
                     urow_v, irow_v, uh_v, ih_v,
                     urows_v, irows_v, out_v, sem):
        wid = lax.axis_index("s") * NUM_CORES + lax.axis_index("c")
        base = wid * BPW

        pltpu.sync_copy(urow_hbm.at[wid], urow_v)
        pltpu.sync_copy(irow_hbm.at[wid], irow_v)
        pltpu.sync_copy(uh_hbm.at[wid], uh_v)
        pltpu.sync_copy(ih_hbm.at[wid], ih_v)

        lane = lax.broadcasted_iota(jnp.int32, (LANES,), 0)
        perms = [lane ^ (1 << k) for k in range(4)]
        dnums = lax.GatherDimensionNumbers(
            offset_dims=(), collapsed_slice_dims=(0,), start_index_map=(0,))

        def _permute(v, idx):
            return lax.gather(
                v, idx[:, None], dnums, slice_sizes=(1,),
                mode=lax.GatherScatterMode.PROMISE_IN_BOUNDS)

        def _allsum(v):
            for p in perms:
                v = v + _permute(v, p)
            return v

        hmask = jnp.int32(-65536)

        for p in range(PASSES):
            copies = []
            for j in range(CPP):
                copies.append(pltpu.async_copy(
                    tab_hbm.at[urow_v.at[p * CPP + j]],
                    urows_v.at[pl.ds(j * CHUNK, CHUNK)], sem))
                copies.append(pltpu.async_copy(
                    tab_hbm.at[irow_v.at[p * CPP + j]],
                    irows_v.at[pl.ds(j * CHUNK, CHUNK)], sem))
            for c in copies:
                c.wait()

            def group_body(g, carry):
                base_row = g * LANES
                chunk_row = p * CPP + g // (CHUNK // LANES)
                chunk_off = (g % (CHUNK // LANES)) * LANES
                uh16 = uh_v[chunk_row, pl.ds(chunk_off, LANES)]
                ih16 = ih_v[chunk_row, pl.ds(chunk_off, LANES)]
                res = jnp.zeros((LANES,), jnp.float32)
                for r in range(LANES):
                    b = base_row + r
                    rsplat = jnp.full((LANES,), r, jnp.int32)
                    uh = _permute(uh16, rsplat)
                    ih = _permute(ih16, rsplat)
                    acc = None
                    for c in range(HDIM // LANES):
                        u0 = urows_v[b, pl.ds(c * LANES, LANES)]
                        u1 = urows_v[b, pl.ds(DIM + c * LANES, LANES)]
                        i0 = irows_v[b, pl.ds(HDIM + c * LANES, LANES)]
                        i1 = irows_v[b, pl.ds(DIM + HDIM + c * LANES, LANES)]
                        ubits = jnp.where(uh == 1, u1, u0)
                        vbits = jnp.where(ih == 1, i1, i0)
                        ulo = plsc.bitcast(ubits << 16, jnp.float32)
                        uhi = plsc.bitcast(ubits & hmask, jnp.float32)
                        vlo = plsc.bitcast(vbits << 16, jnp.float32)
                        vhi = plsc.bitcast(vbits & hmask, jnp.float32)
                        prod = ulo * vlo + uhi * vhi
                        acc = prod if acc is None else acc + prod
                    res = jnp.where(lane == r, _allsum(acc), res)
                out_v[pl.ds(p * EPP + base_row, LANES)] = res
                return carry

            lax.fori_loop(0, EPP // LANES, group_body, 0)

        pltpu.sync_copy(out_v, out_hbm.at[pl.ds(base, BPW)])

    return lightgcn_dot


_SC_KERNEL = _make_sc_kernel()


def _row_half(ids):
    row = (ids // TCW) * HROW + (ids % HROW)
    half = (ids % TCW) // HROW
    return row, half


def kernel(user_ids, item_ids, user_table, item_table):
    uids = user_ids.astype(jnp.int32)
    iids = item_ids.astype(jnp.int32)
    urow, uh = _row_half(uids)
    irow, ih = _row_half(iids)
    shape = (NW, NCHUNK, CHUNK)
    tab = _pack(user_table.T, item_table.T)
    return _SC_KERNEL(urow.reshape(shape), irow.reshape(shape),
                      uh.reshape(shape), ih.reshape(shape), tab)


# speedup vs baseline: 1.8245x; 1.0526x over previous
import functools

import jax
import jax.numpy as jnp
from jax import lax
from jax.experimental import pallas as pl
from jax.experimental.pallas import tpu as pltpu
from jax.experimental.pallas import tpu_sc as plsc

NUM_CORES = 2
NUM_SUBCORES = 16
LANES = 16
NW = NUM_CORES * NUM_SUBCORES
BATCH = 16384
DIM = 64
HDIM = DIM // 2
NROWS = 1000000
BPW = BATCH // NW
CHUNK = 128
NCHUNK = BPW // CHUNK
PASSES = 2
CPP = NCHUNK // PASSES
EPP = BPW // PASSES
TCW = 32768
HROW = TCW // 2
NBLK = (NROWS + TCW - 1) // TCW
PROWS = NBLK * HROW


def _pack_body(u_ref, i_ref, out_ref, xt_ref):
    packs = []
    for src in (u_ref, i_ref):
        bits = lax.bitcast_convert_type(src[...], jnp.int32)
        rne = (bits + 0x7FFF + ((bits >> 16) & 1)) >> 16
        packs.append((rne[HDIM:DIM, :] << 16) | (rne[0:HDIM, :] & 0xFFFF))
    cat = jnp.concatenate(packs, axis=0)
    xt_ref[...] = lax.bitcast_convert_type(cat, jnp.float32)
    catt = xt_ref[...].T
    out_ref[:, 0:DIM] = lax.bitcast_convert_type(
        catt[0:HROW, :], jnp.int32)
    out_ref[:, DIM:2 * DIM] = lax.bitcast_convert_type(
        catt[HROW:TCW, :], jnp.int32)


def _pack(ut, it):
    return pl.pallas_call(
        _pack_body,
        grid=(NBLK,),
        in_specs=[
            pl.BlockSpec((DIM, TCW), lambda i: (0, i)),
            pl.BlockSpec((DIM, TCW), lambda i: (0, i)),
        ],
        out_specs=pl.BlockSpec((HROW, 2 * DIM), lambda i: (i, 0)),
        out_shape=jax.ShapeDtypeStruct((PROWS, 2 * DIM), jnp.int32),
        scratch_shapes=[pltpu.VMEM((DIM, TCW), jnp.float32)],
    )(ut, it)


def _make_sc_kernel():
    mesh = plsc.VectorSubcoreMesh(core_axis_name="c", subcore_axis_name="s")

    @functools.partial(
        pl.kernel,
        mesh=mesh,
        out_type=jax.ShapeDtypeStruct((BATCH,), jnp.float32),
        compiler_params=pltpu.CompilerParams(
            use_tc_tiling_on_sc=False, needs_layout_passes=False),
        scratch_types=[
            pltpu.VMEM((NCHUNK, CHUNK), jnp.int32),
            pltpu.VMEM((NCHUNK, CHUNK), jnp.int32),
            pltpu.VMEM((NCHUNK, CHUNK), jnp.int32),
            pltpu.VMEM((NCHUNK, CHUNK), jnp.int32),
            pltpu.VMEM((EPP, 2 * DIM), jnp.int32),
            pltpu.VMEM((EPP, 2 * DIM), jnp.int32),
            pltpu.VMEM((BPW,), jnp.float32),
            pltpu.SemaphoreType.DMA,
        ],
    )
    def lightgcn_dot(urow_hbm, irow_hbm, uh_hbm, ih_hbm, tab_hbm, out_hbm,
                     urow_v, irow_v, uh_v, ih_v,
                     urows_v, irows_v, out_v, sem):
        wid = lax.axis_index("s") * NUM_CORES + lax.axis_index("c")
        base = wid * BPW

        pltpu.sync_copy(urow_hbm.at[wid], urow_v)
        pltpu.sync_copy(irow_hbm.at[wid], irow_v)
        pltpu.sync_copy(uh_hbm.at[wid], uh_v)
        pltpu.sync_copy(ih_hbm.at[wid], ih_v)

        lane = lax.broadcasted_iota(jnp.int32, (LANES,), 0)
        perms = [lane ^ (1 << k) for k in range(4)]
        dnums = lax.GatherDimensionNumbers(
            offset_dims=(), collapsed_slice_dims=(0,), start_index_map=(0,))

        def _permute(v, idx):
            return lax.gather(
                v, idx[:, None], dnums, slice_sizes=(1,),
                mode=lax.GatherScatterMode.PROMISE_IN_BOUNDS)

        def _allsum(v):
            for p in perms:
                v = v + _permute(v, p)
            return v

        hmask = jnp.int32(-65536)

        for p in range(PASSES):
            copies = []
            for j in range(CPP):
                copies.append(pltpu.async_copy(
                    tab_hbm.at[urow_v.at[p * CPP + j]],
                    urows_v.at[pl.ds(j * CHUNK, CHUNK)], sem))
                copies.append(pltpu.async_copy(
                    tab_hbm.at[irow_v.at[p * CPP + j]],
                    irows_v.at[pl.ds(j * CHUNK, CHUNK)], sem))
            for c in copies:
                c.wait()

            def group_body(g, carry):
                base_row = g * LANES
                chunk_row = p * CPP + g // (CHUNK // LANES)
                chunk_off = (g % (CHUNK // LANES)) * LANES
                uh16 = uh_v[chunk_row, pl.ds(chunk_off, LANES)]
                ih16 = ih_v[chunk_row, pl.ds(chunk_off, LANES)]
                res = jnp.zeros((LANES,), jnp.float32)
                for r in range(LANES):
                    b = base_row + r
                    rsplat = jnp.full((LANES,), r, jnp.int32)
                    uh = _permute(uh16, rsplat)
                    ih = _permute(ih16, rsplat)
                    acc = None
                    for c in range(HDIM // LANES):
                        u0 = urows_v[b, pl.ds(c * LANES, LANES)]
                        u1 = urows_v[b, pl.ds(DIM + c * LANES, LANES)]
                        i0 = irows_v[b, pl.ds(HDIM + c * LANES, LANES)]
                        i1 = irows_v[b, pl.ds(DIM + HDIM + c * LANES, LANES)]
                        ubits = jnp.where(uh == 1, u1, u0)
                        vbits = jnp.where(ih == 1, i1, i0)
                        ulo = plsc.bitcast(ubits << 16, jnp.float32)
                        uhi = plsc.bitcast(ubits & hmask, jnp.float32)
                        vlo = plsc.bitcast(vbits << 16, jnp.float32)
                        vhi = plsc.bitcast(vbits & hmask, jnp.float32)
                        prod = ulo * vlo + uhi * vhi
                        acc = prod if acc is None else acc + prod
                    res = jnp.where(lane == r, _allsum(acc), res)
                out_v[pl.ds(p * EPP + base_row, LANES)] = res
                return carry

            lax.fori_loop(0, EPP // LANES, group_body, 0)

        pltpu.sync_copy(out_v, out_hbm.at[pl.ds(base, BPW)])

    return lightgcn_dot


_SC_KERNEL = _make_sc_kernel()


def _row_half(ids):
    row = (ids // TCW) * HROW + (ids % HROW)
    half = (ids % TCW) // HROW
    return row, half


def kernel(user_ids, item_ids, user_table, item_table):
    uids = user_ids.astype(jnp.int32)
    iids = item_ids.astype(jnp.int32)
    urow, uh = _row_half(uids)
    irow, ih = _row_half(iids)
    shape = (NW, NCHUNK, CHUNK)
    tab = _pack(user_table.T, item_table.T)
    return _SC_KERNEL(urow.reshape(shape), irow.reshape(shape),
                      uh.reshape(shape), ih.reshape(shape), tab)
